# Initial kernel scaffold; baseline (speedup 1.0000x reference)
import jax
import jax.numpy as jnp
from jax.experimental import pallas as pl


def kernel(x, node_types, W):
    raise NotImplementedError("write your pallas kernel here")


# speedup vs baseline: 5.3372x; 5.3372x over previous
import functools

import jax
import jax.numpy as jnp
from jax import lax
from jax.experimental import pallas as pl
from jax.experimental.pallas import tpu as pltpu
from jax.experimental.pallas import tpu_sc as plsc

N, V, DIM = 4096, 200, 64
NC, NS = 2, 16
B_PER_W = 8
N_WORKERS = V // B_PER_W
LANES = 16
TN = 128


def _sc_gather(W2, idx_hi, lo16):
    mesh = plsc.VectorSubcoreMesh(core_axis_name="c", subcore_axis_name="s")

    @functools.partial(
        pl.kernel,
        mesh=mesh,
        out_type=jax.ShapeDtypeStruct((V, DIM), jnp.float32),
        scratch_types=[
            pltpu.VMEM((B_PER_W,), jnp.int32),
            pltpu.VMEM((B_PER_W, LANES), jnp.int32),
            pltpu.VMEM((B_PER_W, 2 * DIM), jnp.float32),
            pltpu.VMEM((B_PER_W, DIM), jnp.float32),
            pltpu.SemaphoreType.DMA,
        ],
    )
    def gather_kernel(w_hbm, idxhi_hbm, lo16_hbm, out_hbm,
                      idx_v, lo_v, rows_v, out_v, sem):
        wid = lax.axis_index("s") * NC + lax.axis_index("c")

        @pl.when(wid < N_WORKERS)
        def _():
            base = wid * B_PER_W
            pltpu.sync_copy(idxhi_hbm.at[pl.ds(base, B_PER_W)], idx_v)
            pltpu.sync_copy(lo16_hbm.at[pl.ds(base, B_PER_W)], lo_v)
            pltpu.async_copy(w_hbm.at[idx_v], rows_v, sem).wait()
            for r in range(B_PER_W):
                m = lo_v[r, :] > 0
                for j in range(DIM // LANES):
                    low = rows_v[r, pl.ds(j * LANES, LANES)]
                    high = rows_v[r, pl.ds(DIM + j * LANES, LANES)]
                    out_v[r, pl.ds(j * LANES, LANES)] = jnp.where(m, high, low)
            pltpu.sync_copy(out_v, out_hbm.at[pl.ds(base, B_PER_W)])

    return gather_kernel(W2, idx_hi, lo16)


def _add_body(x_ref, e_ref, o_ref):
    o_ref[...] = x_ref[...] + e_ref[...]


def _tc_add(x2d, e2d):
    return pl.pallas_call(
        _add_body,
        grid=(N // TN,),
        in_specs=[
            pl.BlockSpec((TN, V * DIM), lambda i: (i, 0)),
            pl.BlockSpec((1, V * DIM), lambda i: (0, 0)),
        ],
        out_specs=pl.BlockSpec((TN, V * DIM), lambda i: (i, 0)),
        out_shape=jax.ShapeDtypeStruct((N, V * DIM), jnp.float32),
    )(x2d, e2d)


@jax.jit
def kernel(x, node_types, W):
    idx = node_types.reshape(V)
    idx_hi = idx >> 1
    lo16 = jnp.broadcast_to((idx & 1).reshape(V, 1), (V, LANES))
    embeds = _sc_gather(W.reshape(-1, 2 * DIM), idx_hi, lo16)
    out2d = _tc_add(x.reshape(N, V * DIM), embeds.reshape(1, V * DIM))
    return out2d.reshape(N, V, DIM)


# speedup vs baseline: 14.1484x; 2.6509x over previous
import functools

import jax
import jax.numpy as jnp
from jax import lax
from jax.experimental import pallas as pl
from jax.experimental.pallas import tpu as pltpu
from jax.experimental.pallas import tpu_sc as plsc

N, V, DIM = 4096, 200, 64
NC, NS = 2, 16
B_PER_W = 8
N_WORKERS = V // B_PER_W
LANES = 16
BV = 8


def _sc_gather(W2, idx_hi, lo16):
    mesh = plsc.VectorSubcoreMesh(core_axis_name="c", subcore_axis_name="s")

    @functools.partial(
        pl.kernel,
        mesh=mesh,
        out_type=jax.ShapeDtypeStruct((V, DIM), jnp.float32),
        scratch_types=[
            pltpu.VMEM((B_PER_W,), jnp.int32),
            pltpu.VMEM((B_PER_W, LANES), jnp.int32),
            pltpu.VMEM((B_PER_W, 2 * DIM), jnp.float32),
            pltpu.VMEM((B_PER_W, DIM), jnp.float32),
            pltpu.SemaphoreType.DMA,
        ],
    )
    def gather_kernel(w_hbm, idxhi_hbm, lo16_hbm, out_hbm,
                      idx_v, lo_v, rows_v, out_v, sem):
        wid = lax.axis_index("s") * NC + lax.axis_index("c")

        @pl.when(wid < N_WORKERS)
        def _():
            base = wid * B_PER_W
            pltpu.sync_copy(idxhi_hbm.at[pl.ds(base, B_PER_W)], idx_v)
            pltpu.sync_copy(lo16_hbm.at[pl.ds(base, B_PER_W)], lo_v)
            pltpu.async_copy(w_hbm.at[idx_v], rows_v, sem).wait()
            for r in range(B_PER_W):
                m = lo_v[r, :] > 0
                for j in range(DIM // LANES):
                    low = rows_v[r, pl.ds(j * LANES, LANES)]
                    high = rows_v[r, pl.ds(DIM + j * LANES, LANES)]
                    out_v[r, pl.ds(j * LANES, LANES)] = jnp.where(m, high, low)
            pltpu.sync_copy(out_v, out_hbm.at[pl.ds(base, B_PER_W)])

    return gather_kernel(W2, idx_hi, lo16)


def _add_body(x_ref, e_ref, o_ref):
    o_ref[...] = x_ref[...] + e_ref[...]


def _tc_add(xt, e3):
    return pl.pallas_call(
        _add_body,
        grid=(V // BV,),
        in_specs=[
            pl.BlockSpec((BV, DIM, N), lambda i: (i, 0, 0)),
            pl.BlockSpec((BV, DIM, 1), lambda i: (i, 0, 0)),
        ],
        out_specs=pl.BlockSpec((BV, DIM, N), lambda i: (i, 0, 0)),
        out_shape=jax.ShapeDtypeStruct((V, DIM, N), jnp.float32),
    )(xt, e3)


@jax.jit
def kernel(x, node_types, W):
    idx = node_types.reshape(V)
    idx_hi = idx >> 1
    lo16 = jnp.broadcast_to((idx & 1).reshape(V, 1), (V, LANES))
    embeds = _sc_gather(W.reshape(-1, 2 * DIM), idx_hi, lo16)
    xt = jnp.transpose(x, (1, 2, 0))
    out_t = _tc_add(xt, embeds.reshape(V, DIM, 1))
    return jnp.transpose(out_t, (2, 0, 1))
